# baseline (speedup 1.0000x reference)
import jax
import jax.numpy as jnp
from jax.experimental import pallas as pl

BS, N, XYZ = 2, 2048, 4
K = 32
BQ = 128


def _d2_block(x_ref, o_ref):
    x = x_ref[0]
    y = x * x
    y2 = y.reshape(BQ * (N * XYZ // 128), 128)
    l = jax.lax.broadcasted_iota(jnp.int32, (128, 32), 0)
    g = jax.lax.broadcasted_iota(jnp.int32, (128, 32), 1)
    c4 = (l // 4 == g).astype(jnp.float32)
    d2 = jax.lax.dot_general(y2, c4, (((1,), (0,)), ((), ())),
                             precision=jax.lax.Precision.HIGHEST,
                             preferred_element_type=jnp.float32)
    o_ref[0] = d2.reshape(BQ, N * XYZ // 128, 32)


def _pairwise_d2(pairs_flat):
    out = pl.pallas_call(
        _d2_block,
        grid=(BS, N // BQ),
        in_specs=[pl.BlockSpec((1, BQ, N * XYZ), lambda b, q: (b, q, 0))],
        out_specs=pl.BlockSpec((1, BQ, N * XYZ // 128, 32),
                               lambda b, q: (b, q, 0, 0)),
        out_shape=jax.ShapeDtypeStruct((BS, N, N * XYZ // 128, 32),
                                       jnp.float32),
    )(pairs_flat)
    return out.reshape(BS, N, N)


def _swish(x):
    return x * jax.nn.sigmoid(x)


def kernel(pairs_abq, vals, mask, W1, b1, W2, b2, W3, b3, Wl, bl):
    bs, n = vals.shape[:2]
    pairs_flat = pairs_abq.reshape(bs, n, n * XYZ)
    d2 = _pairwise_d2(pairs_flat)
    d2 = jnp.where(jnp.broadcast_to(mask[:, None, :], d2.shape), d2, 1e16)
    _, nbhd_idx = jax.lax.top_k(-d2, K)
    B = jnp.arange(bs)[:, None, None]
    M = jnp.arange(n)[None, :, None]
    nbhd_abq = pairs_abq[B, M, nbhd_idx]
    nbhd_vals = vals[B, nbhd_idx]
    nbhd_mask = mask[B, nbhd_idx]
    h = _swish(nbhd_abq @ W1 + b1)
    h = _swish(h @ W2 + b2)
    kw = _swish(h @ W3 + b3)
    kw_m = jnp.where(nbhd_mask[..., None], kw, 0.0)
    vals_m = jnp.where(nbhd_mask[..., None], nbhd_vals, 0.0)
    partial = jnp.einsum('bmkc,bmko->bmco', vals_m, kw_m).reshape(bs, n, -1)
    convolved = partial @ Wl + bl
    convolved = jnp.where(mask[..., None], convolved, 0.0)
    return (pairs_abq, convolved, mask)
